# Initial kernel scaffold; baseline (speedup 1.0000x reference)
import jax
import jax.numpy as jnp
from jax.experimental import pallas as pl


def kernel(z, cell_type, means, log_vars):
    raise NotImplementedError("write your pallas kernel here")


# speedup vs baseline: 2.3008x; 2.3008x over previous
import functools
import math

import jax
import jax.numpy as jnp
from jax import lax
from jax.experimental import pallas as pl
from jax.experimental.pallas import tpu as pltpu
from jax.experimental.pallas import tpu_sc as plsc

D = 128
B = 16384
NC = 2
NS = 16
NW = NC * NS
ROWS_PER_W = B // NW
CH = 128
N_CHUNKS = ROWS_PER_W // CH
LANES = 16
VPR = D // LANES
LOG2PI = math.log(2.0 * math.pi)
CONST = D * LOG2PI

_mesh = plsc.VectorSubcoreMesh(core_axis_name="c", subcore_axis_name="s")

_GDN = lax.GatherDimensionNumbers(
    offset_dims=(), collapsed_slice_dims=(0,), start_index_map=(0,))


def _shuffle(x, perm):
    return lax.gather(x, perm[:, None], dimension_numbers=_GDN,
                      slice_sizes=(1,),
                      mode=lax.GatherScatterMode.PROMISE_IN_BOUNDS)


def _hsum(x, perms):
    for p in perms:
        x = x + _shuffle(x, p)
    return x


@functools.partial(
    pl.kernel,
    mesh=_mesh,
    out_type=jax.ShapeDtypeStruct((B,), jnp.float32),
    scratch_types=[
        pltpu.VMEM((CH,), jnp.int32),
        pltpu.VMEM((CH, D), jnp.float32),
        pltpu.VMEM((CH, D), jnp.float32),
        pltpu.VMEM((CH, D), jnp.float32),
        pltpu.VMEM((CH,), jnp.float32),
        pltpu.SemaphoreType.DMA,
    ],
)
def _sc_body(z_hbm, ct_hbm, means_hbm, lv_hbm, out_hbm,
             idx_v, z_v, m_v, lv_v, out_v, sem):
    wid = lax.axis_index("s") * NC + lax.axis_index("c")
    lane = lax.iota(jnp.int32, LANES)
    perms = [lane ^ sh for sh in (8, 4, 2, 1)]
    for chunk in range(N_CHUNKS):
        base = wid * ROWS_PER_W + chunk * CH
        pltpu.sync_copy(ct_hbm.at[pl.ds(base, CH)], idx_v)
        cm = pltpu.async_copy(means_hbm.at[idx_v], m_v, sem)
        cl = pltpu.async_copy(lv_hbm.at[idx_v], lv_v, sem)
        cz = pltpu.async_copy(z_hbm.at[pl.ds(base, CH)], z_v, sem)
        cm.wait()
        cl.wait()
        cz.wait()
        for g in range(CH // LANES):
            def row_body(j, outvec, g=g):
                r = g * LANES + j
                s = jnp.zeros((LANES,), jnp.float32)
                for k in range(VPR):
                    zv = z_v[r, pl.ds(k * LANES, LANES)]
                    mv = m_v[r, pl.ds(k * LANES, LANES)]
                    lvv = lv_v[r, pl.ds(k * LANES, LANES)]
                    diff = zv - mv
                    s = s + (diff * diff * jnp.exp(-lvv) + lvv)
                tot = -0.5 * (_hsum(s, perms) + CONST)
                return jnp.where(lane == j, tot, outvec)

            outvec = lax.fori_loop(0, LANES, row_body,
                                   jnp.zeros((LANES,), jnp.float32))
            out_v[pl.ds(g * LANES, LANES)] = outvec
        pltpu.sync_copy(out_v, out_hbm.at[pl.ds(base, CH)])


def kernel(z, cell_type, means, log_vars):
    ct = cell_type.astype(jnp.int32)
    return _sc_body(z, ct, means, log_vars)


# speedup vs baseline: 2.6830x; 1.1661x over previous
import functools
import math

import jax
import jax.numpy as jnp
from jax import lax
from jax.experimental import pallas as pl
from jax.experimental.pallas import tpu as pltpu
from jax.experimental.pallas import tpu_sc as plsc

D = 128
B = 16384
NC = 2
NS = 16
NW = NC * NS
ROWS_PER_W = B // NW
CH = 128
N_CHUNKS = ROWS_PER_W // CH
LANES = 16
VPR = D // LANES
LOG2PI = math.log(2.0 * math.pi)
CONST = D * LOG2PI

_mesh = plsc.VectorSubcoreMesh(core_axis_name="c", subcore_axis_name="s")

_GDN = lax.GatherDimensionNumbers(
    offset_dims=(), collapsed_slice_dims=(0,), start_index_map=(0,))


def _shuffle(x, perm):
    return lax.gather(x, perm[:, None], dimension_numbers=_GDN,
                      slice_sizes=(1,),
                      mode=lax.GatherScatterMode.PROMISE_IN_BOUNDS)


def _hsum(x, perms):
    for p in perms:
        x = x + _shuffle(x, p)
    return x


@functools.partial(
    pl.kernel,
    mesh=_mesh,
    out_type=jax.ShapeDtypeStruct((B,), jnp.float32),
    scratch_types=[
        pltpu.VMEM((ROWS_PER_W,), jnp.int32),
        pltpu.VMEM((2, CH, D), jnp.float32),
        pltpu.VMEM((2, CH, D), jnp.float32),
        pltpu.VMEM((2, CH, D), jnp.float32),
        pltpu.VMEM((ROWS_PER_W,), jnp.float32),
        pltpu.SemaphoreType.DMA,
        pltpu.SemaphoreType.DMA,
    ],
)
def _sc_body(z_hbm, ct_hbm, means_hbm, lv_hbm, out_hbm,
             idx_v, z_v, m_v, lv_v, out_v, sem0, sem1):
    wid = lax.axis_index("s") * NC + lax.axis_index("c")
    wbase = wid * ROWS_PER_W
    lane = lax.iota(jnp.int32, LANES)
    perms = [lane ^ sh for sh in (8, 4, 2, 1)]
    sems = (sem0, sem1)

    pltpu.sync_copy(ct_hbm.at[pl.ds(wbase, ROWS_PER_W)], idx_v)

    def issue(c):
        b = c % 2
        ii = idx_v.at[pl.ds(c * CH, CH)]
        return (
            pltpu.async_copy(means_hbm.at[ii], m_v.at[b], sems[b]),
            pltpu.async_copy(lv_hbm.at[ii], lv_v.at[b], sems[b]),
            pltpu.async_copy(z_hbm.at[pl.ds(wbase + c * CH, CH)],
                             z_v.at[b], sems[b]),
        )

    inflight = {0: issue(0)}
    for chunk in range(N_CHUNKS):
        b = chunk % 2
        if chunk + 1 < N_CHUNKS:
            inflight[chunk + 1] = issue(chunk + 1)
        for cp in inflight.pop(chunk):
            cp.wait()
        zb, mb, lvb = z_v.at[b], m_v.at[b], lv_v.at[b]
        for g in range(CH // LANES):
            def row_body(j, outvec, g=g):
                r = g * LANES + j
                s = jnp.zeros((LANES,), jnp.float32)
                for k in range(VPR):
                    zv = zb[r, pl.ds(k * LANES, LANES)]
                    mv = mb[r, pl.ds(k * LANES, LANES)]
                    lvv = lvb[r, pl.ds(k * LANES, LANES)]
                    diff = zv - mv
                    s = s + (diff * diff * jnp.exp(-lvv) + lvv)
                tot = -0.5 * (_hsum(s, perms) + CONST)
                return jnp.where(lane == j, tot, outvec)

            outvec = lax.fori_loop(0, LANES, row_body,
                                   jnp.zeros((LANES,), jnp.float32))
            out_v[pl.ds(chunk * CH + g * LANES, LANES)] = outvec
    pltpu.sync_copy(out_v, out_hbm.at[pl.ds(wbase, ROWS_PER_W)])


def kernel(z, cell_type, means, log_vars):
    ct = cell_type.astype(jnp.int32)
    return _sc_body(z, ct, means, log_vars)


# speedup vs baseline: 3.2587x; 1.2146x over previous
import functools
import math

import jax
import jax.numpy as jnp
from jax import lax
from jax.experimental import pallas as pl
from jax.experimental.pallas import tpu as pltpu
from jax.experimental.pallas import tpu_sc as plsc

D = 128
B = 16384
NC = 2
NS = 16
NW = NC * NS
ROWS_PER_W = B // NW
CH = 128
N_CHUNKS = ROWS_PER_W // CH
LANES = 16
VPR = D // LANES
LOG2PI = math.log(2.0 * math.pi)
CONST = D * LOG2PI

_mesh = plsc.VectorSubcoreMesh(core_axis_name="c", subcore_axis_name="s")

_GDN = lax.GatherDimensionNumbers(
    offset_dims=(), collapsed_slice_dims=(0,), start_index_map=(0,))


def _shuffle(x, perm):
    return lax.gather(x, perm[:, None], dimension_numbers=_GDN,
                      slice_sizes=(1,),
                      mode=lax.GatherScatterMode.PROMISE_IN_BOUNDS)


def _hsum(x, perms):
    for p in perms:
        x = x + _shuffle(x, p)
    return x


@functools.partial(
    pl.kernel,
    mesh=_mesh,
    out_type=jax.ShapeDtypeStruct((B,), jnp.float32),
    scratch_types=[
        pltpu.VMEM((ROWS_PER_W,), jnp.int32),
        pltpu.VMEM((2, CH, D), jnp.float32),
        pltpu.VMEM((2, CH, 2 * D), jnp.float32),
        pltpu.VMEM((ROWS_PER_W,), jnp.float32),
        pltpu.SemaphoreType.DMA,
        pltpu.SemaphoreType.DMA,
    ],
)
def _sc_body(z_hbm, ct_hbm, tab_hbm, out_hbm,
             idx_v, z_v, t_v, out_v, sem0, sem1):
    wid = lax.axis_index("s") * NC + lax.axis_index("c")
    wbase = wid * ROWS_PER_W
    lane = lax.iota(jnp.int32, LANES)
    perms = [lane ^ sh for sh in (8, 4, 2, 1)]
    sems = (sem0, sem1)

    pltpu.sync_copy(ct_hbm.at[pl.ds(wbase, ROWS_PER_W)], idx_v)

    def issue(c):
        b = c % 2
        ii = idx_v.at[pl.ds(c * CH, CH)]
        return (
            pltpu.async_copy(tab_hbm.at[ii], t_v.at[b], sems[b]),
            pltpu.async_copy(z_hbm.at[pl.ds(wbase + c * CH, CH)],
                             z_v.at[b], sems[b]),
        )

    inflight = {0: issue(0)}
    for chunk in range(N_CHUNKS):
        b = chunk % 2
        if chunk + 1 < N_CHUNKS:
            inflight[chunk + 1] = issue(chunk + 1)
        for cp in inflight.pop(chunk):
            cp.wait()
        zb, tb = z_v.at[b], t_v.at[b]

        def group_body(g, _, chunk=chunk, zb=zb, tb=tb):
            def row_body(j, outvec):
                r = g * LANES + j
                s = jnp.zeros((LANES,), jnp.float32)
                for k in range(VPR):
                    zv = zb[r, pl.ds(k * LANES, LANES)]
                    mv = tb[r, pl.ds(k * LANES, LANES)]
                    lvv = tb[r, pl.ds(D + k * LANES, LANES)]
                    diff = zv - mv
                    s = s + (diff * diff * jnp.exp(-lvv) + lvv)
                tot = -0.5 * (_hsum(s, perms) + CONST)
                return jnp.where(lane == j, tot, outvec)

            outvec = lax.fori_loop(0, LANES, row_body,
                                   jnp.zeros((LANES,), jnp.float32))
            out_v[pl.ds(chunk * CH + g * LANES, LANES)] = outvec
            return 0

        lax.fori_loop(0, CH // LANES, group_body, 0)
    pltpu.sync_copy(out_v, out_hbm.at[pl.ds(wbase, ROWS_PER_W)])


def kernel(z, cell_type, means, log_vars):
    ct = cell_type.astype(jnp.int32)
    tab = jnp.concatenate([means, log_vars], axis=1)
    return _sc_body(z, ct, tab)
